# baseline (speedup 1.0000x reference)
import functools

import jax
import jax.numpy as jnp
from jax.experimental import pallas as pl
from jax.experimental.pallas import tpu as pltpu

_NEG_BIG = -1e30


def _fused_head_kernel(hw, n_groups, x_ref, s_ref, w_ref, b_ref, lbl_ref,
                       loss_ref, hit1_ref, hit5_ref):
    pooled_t = jax.lax.dot_general(
        x_ref[...], s_ref[...], (((1,), (0,)), ((), ())),
        precision=jax.lax.Precision.HIGHEST,
        preferred_element_type=jnp.float32)
    tile_n = pooled_t.shape[0] // n_groups
    pooled = pooled_t.reshape(tile_n, n_groups * 128) * (1.0 / hw)

    logits = jnp.dot(pooled, w_ref[...],
                     preferred_element_type=jnp.float32) + b_ref[...]

    m = jnp.max(logits, axis=1, keepdims=True)
    lse = m + jnp.log(jnp.sum(jnp.exp(logits - m), axis=1, keepdims=True))
    tn, kp = logits.shape
    cls_iota = jax.lax.broadcasted_iota(jnp.int32, (tn, kp), 1)
    lbl = lbl_ref[...]
    picked = jnp.sum(jnp.where(cls_iota == lbl, logits, 0.0),
                     axis=1, keepdims=True)
    loss_ref[...] = lse - picked

    n_greater = jnp.sum((logits > picked).astype(jnp.float32),
                        axis=1, keepdims=True)
    n_eq_before = jnp.sum(((logits == picked) & (cls_iota < lbl))
                          .astype(jnp.float32), axis=1, keepdims=True)
    rank = n_greater + n_eq_before
    hit1_ref[...] = (rank < 1.0).astype(jnp.float32)
    hit5_ref[...] = (rank < 5.0).astype(jnp.float32)


def kernel(x, w, b, gt_label):
    N, C, H, W = x.shape
    K = w.shape[1]
    HW = H * W
    assert C % 128 == 0
    n_groups = C // 128
    GW = 128 * HW

    x2 = x.reshape(N * n_groups, GW)

    seg = (jax.lax.broadcasted_iota(jnp.int32, (GW, 128), 0) // HW ==
           jax.lax.broadcasted_iota(jnp.int32, (GW, 128), 1)
           ).astype(jnp.float32)

    K_pad = max(128, ((K + 127) // 128) * 128)
    w_pad = jnp.pad(w, ((0, 0), (0, K_pad - K)))
    b_pad = jnp.pad(b.reshape(1, K), ((0, 0), (0, K_pad - K)),
                    constant_values=_NEG_BIG)
    lbl2 = gt_label.astype(jnp.int32).reshape(N, 1)

    TILE_N = min(N, 16)
    grid = (pl.cdiv(N, TILE_N),)

    loss, hit1, hit5 = pl.pallas_call(
        functools.partial(_fused_head_kernel, HW, n_groups),
        out_shape=(
            jax.ShapeDtypeStruct((N, 1), jnp.float32),
            jax.ShapeDtypeStruct((N, 1), jnp.float32),
            jax.ShapeDtypeStruct((N, 1), jnp.float32),
        ),
        grid=grid,
        in_specs=[
            pl.BlockSpec((TILE_N * n_groups, GW), lambda i: (i, 0)),
            pl.BlockSpec((GW, 128), lambda i: (0, 0)),
            pl.BlockSpec((C, K_pad), lambda i: (0, 0)),
            pl.BlockSpec((1, K_pad), lambda i: (0, 0)),
            pl.BlockSpec((TILE_N, 1), lambda i: (i, 0)),
        ],
        out_specs=(
            pl.BlockSpec((TILE_N, 1), lambda i: (i, 0)),
            pl.BlockSpec((TILE_N, 1), lambda i: (i, 0)),
            pl.BlockSpec((TILE_N, 1), lambda i: (i, 0)),
        ),
        compiler_params=pltpu.CompilerParams(
            dimension_semantics=("parallel",),
            vmem_limit_bytes=48 * 1024 * 1024,
        ),
    )(x2, seg, w_pad, b_pad, lbl2)

    return {
        "loss": jnp.mean(loss),
        "accuracy": {
            "top-1": jnp.mean(hit1) * 100.0,
            "top-5": jnp.mean(hit5) * 100.0,
        },
    }


# speedup vs baseline: 25.7017x; 25.7017x over previous
import jax
import jax.numpy as jnp
from jax.experimental import pallas as pl
from jax.experimental.pallas import tpu as pltpu


def _fused_head_kernel(x_ref, wt_ref, b_ref, lbl_ref,
                       loss_ref, hit1_ref, hit5_ref):
    x = x_ref[...]
    hw = x.shape[0]
    pooled = jnp.sum(x, axis=0) * (1.0 / hw)

    logits = jax.lax.dot_general(
        pooled, wt_ref[...], (((1,), (1,)), ((), ())),
        preferred_element_type=jnp.float32) + b_ref[...]

    m = jnp.max(logits, axis=1, keepdims=True)
    lse = m + jnp.log(jnp.sum(jnp.exp(logits - m), axis=1, keepdims=True))
    tn, k = logits.shape
    cls_iota = jax.lax.broadcasted_iota(jnp.int32, (tn, k), 1)
    lbl = lbl_ref[...]
    picked = jnp.sum(jnp.where(cls_iota == lbl, logits, 0.0),
                     axis=1, keepdims=True)
    loss_ref[...] = lse - picked

    n_greater = jnp.sum((logits > picked).astype(jnp.float32),
                        axis=1, keepdims=True)
    n_eq_before = jnp.sum(((logits == picked) & (cls_iota < lbl))
                          .astype(jnp.float32), axis=1, keepdims=True)
    rank = n_greater + n_eq_before
    hit1_ref[...] = (rank < 1.0).astype(jnp.float32)
    hit5_ref[...] = (rank < 5.0).astype(jnp.float32)


def kernel(x, w, b, gt_label):
    N, C, H, W = x.shape
    K = w.shape[1]
    HW = H * W

    xt = jnp.transpose(x, (2, 3, 0, 1)).reshape(HW, N, C)
    wt = jnp.transpose(w)
    b2 = b.reshape(1, K)
    lbl2 = gt_label.astype(jnp.int32).reshape(N, 1)

    TILE_N = min(N, 16)
    grid = (pl.cdiv(N, TILE_N),)

    loss, hit1, hit5 = pl.pallas_call(
        _fused_head_kernel,
        out_shape=(
            jax.ShapeDtypeStruct((N, 1), jnp.float32),
            jax.ShapeDtypeStruct((N, 1), jnp.float32),
            jax.ShapeDtypeStruct((N, 1), jnp.float32),
        ),
        grid=grid,
        in_specs=[
            pl.BlockSpec((HW, TILE_N, C), lambda i: (0, i, 0)),
            pl.BlockSpec((K, C), lambda i: (0, 0)),
            pl.BlockSpec((1, K), lambda i: (0, 0)),
            pl.BlockSpec((TILE_N, 1), lambda i: (i, 0)),
        ],
        out_specs=(
            pl.BlockSpec((TILE_N, 1), lambda i: (i, 0)),
            pl.BlockSpec((TILE_N, 1), lambda i: (i, 0)),
            pl.BlockSpec((TILE_N, 1), lambda i: (i, 0)),
        ),
        compiler_params=pltpu.CompilerParams(
            dimension_semantics=("parallel",),
            vmem_limit_bytes=48 * 1024 * 1024,
        ),
    )(xt, wt, b2, lbl2)

    return {
        "loss": jnp.mean(loss),
        "accuracy": {
            "top-1": jnp.mean(hit1) * 100.0,
            "top-5": jnp.mean(hit5) * 100.0,
        },
    }


# speedup vs baseline: 28.5603x; 1.1112x over previous
import jax
import jax.numpy as jnp
from jax.experimental import pallas as pl
from jax.experimental.pallas import tpu as pltpu


def _fused_head_kernel(x_ref, wt_ref, b_ref, lbl_ref,
                       loss_ref, hit1_ref, hit5_ref):
    x = x_ref[...]
    hw = x.shape[0]
    pooled = jnp.sum(x, axis=0) * (1.0 / hw)

    logits = jax.lax.dot_general(
        pooled, wt_ref[...], (((1,), (1,)), ((), ())),
        preferred_element_type=jnp.float32) + b_ref[...]

    m = jnp.max(logits, axis=1, keepdims=True)
    lse = m + jnp.log(jnp.sum(jnp.exp(logits - m), axis=1, keepdims=True))
    tn, k = logits.shape
    cls_iota = jax.lax.broadcasted_iota(jnp.int32, (tn, k), 1)
    lbl = lbl_ref[...]
    picked = jnp.sum(jnp.where(cls_iota == lbl, logits, 0.0),
                     axis=1, keepdims=True)
    loss_ref[...] = lse - picked

    n_greater = jnp.sum((logits > picked).astype(jnp.float32),
                        axis=1, keepdims=True)
    n_eq_before = jnp.sum(((logits == picked) & (cls_iota < lbl))
                          .astype(jnp.float32), axis=1, keepdims=True)
    rank = n_greater + n_eq_before
    hit1_ref[...] = (rank < 1.0).astype(jnp.float32)
    hit5_ref[...] = (rank < 5.0).astype(jnp.float32)


def kernel(x, w, b, gt_label):
    N, C, H, W = x.shape
    K = w.shape[1]
    HW = H * W

    xt = jnp.transpose(x, (2, 3, 0, 1)).reshape(HW, N, C)
    wt = jnp.transpose(w)
    b2 = b.reshape(1, K)
    lbl2 = gt_label.astype(jnp.int32).reshape(N, 1)

    TILE_N = min(N, 32)
    grid = (pl.cdiv(N, TILE_N),)

    loss, hit1, hit5 = pl.pallas_call(
        _fused_head_kernel,
        out_shape=(
            jax.ShapeDtypeStruct((N, 1), jnp.float32),
            jax.ShapeDtypeStruct((N, 1), jnp.float32),
            jax.ShapeDtypeStruct((N, 1), jnp.float32),
        ),
        grid=grid,
        in_specs=[
            pl.BlockSpec((HW, TILE_N, C), lambda i: (0, i, 0)),
            pl.BlockSpec((K, C), lambda i: (0, 0)),
            pl.BlockSpec((1, K), lambda i: (0, 0)),
            pl.BlockSpec((TILE_N, 1), lambda i: (i, 0)),
        ],
        out_specs=(
            pl.BlockSpec((TILE_N, 1), lambda i: (i, 0)),
            pl.BlockSpec((TILE_N, 1), lambda i: (i, 0)),
            pl.BlockSpec((TILE_N, 1), lambda i: (i, 0)),
        ),
        compiler_params=pltpu.CompilerParams(
            dimension_semantics=("parallel",),
            vmem_limit_bytes=48 * 1024 * 1024,
        ),
    )(xt, wt, b2, lbl2)

    return {
        "loss": jnp.mean(loss),
        "accuracy": {
            "top-1": jnp.mean(hit1) * 100.0,
            "top-5": jnp.mean(hit5) * 100.0,
        },
    }
